# baseline (speedup 1.0000x reference)
import jax
import jax.numpy as jnp
from jax import lax
from jax.experimental import pallas as pl
from jax.experimental.pallas import tpu as pltpu
from jax.experimental.pallas import tpu_sc as plsc

_V = 100000
_F = 128
_FO = 128
_K = 3
_POOL = 4
_VP = _V // _POOL
_NNZ = 600000
_HALF = _F // 2

_E = 128
_RT = 296
_NNZ_PAD = 16 * _RT * _E
_G = 4
_NB = _RT // 8
_NR = 16 * _RT + _G

_RPAD = 25600
_RPT = _RPAD // 16
_ZCH = 200
_ZB = 40


def _pool_body(x_ref, o_ref):
    o_ref[...] = jnp.mean(x_ref[...], axis=1)


def _pool(x4):
    vb = 1000
    return pl.pallas_call(
        _pool_body,
        grid=(_VP // vb,),
        in_specs=[pl.BlockSpec((vb, _POOL, _F), lambda i: (i, 0, 0))],
        out_specs=pl.BlockSpec((vb, _F), lambda i: (i, 0)),
        out_shape=jax.ShapeDtypeStruct((_VP, _F), jnp.float32),
    )(x4)


def _sc_body(x0i, cols2, rows2, vals2, x1h, y2h,
             colsS, rowsS, valsS, idxT, gath, zbuf, acc,
             sem_st, sg0, sg1, ss0, ss1):
    c = lax.axis_index("c")
    s = lax.axis_index("s")
    sg = [sg0, sg1]
    ss = [ss0, ss1]
    row0_tile = s * _RT

    def zero_zbuf(i, _):
        for q in range(4):
            zbuf[i, pl.ds(q * 16, 16)] = jnp.zeros((16,), jnp.float32)
        return 0

    lax.fori_loop(0, _ZB, zero_zbuf, 0)

    def one_pass(table, out, mul, off):
        def zloop(j, _):
            pltpu.sync_copy(zbuf, acc.at[pl.ds(s * _RPT + j * _ZB, _ZB)])
            return 0

        lax.fori_loop(0, _RPT // _ZB, zloop, 0)
        plsc.subcore_barrier()

        def transform(gbuf, grow, p1):
            for q in range(_E // 16):
                sl = pl.ds(q * 16, 16)
                if mul == 2:
                    idxT[p1, sl] = colsS[gbuf, grow, sl] * 2 + off
                else:
                    idxT[p1, sl] = colsS[gbuf, grow, sl] + off

        def start_gather(p1):
            return pltpu.async_copy(table.at[idxT.at[p1]], gath.at[p1],
                                    sg[p1])

        def wait_gather_rec(p1):
            pltpu.make_async_copy(
                table.at[pl.ds(0, _E)], gath.at[p1], sg[p1]).wait()

        def start_scatter(gbuf, grow, p1):
            return pltpu.async_copy(gath.at[p1], acc.at[rowsS.at[gbuf, grow]],
                                    ss[p1], add=True)

        def wait_scatter_rec(p1):
            pltpu.make_async_copy(
                table.at[pl.ds(0, _E)], gath.at[p1], ss[p1]).wait()

        def stage_fire(gbuf, r0):
            pltpu.async_copy(cols2.at[pl.ds(r0, _G)], colsS.at[gbuf], sem_st)
            pltpu.async_copy(rows2.at[pl.ds(r0, _G)], rowsS.at[gbuf], sem_st)
            pltpu.async_copy(vals2.at[pl.ds(r0, _G)], valsS.at[gbuf], sem_st)

        def stage_drain():
            for ref, hbm in ((colsS, cols2), (rowsS, rows2), (valsS, vals2)):
                pltpu.make_async_copy(
                    hbm.at[pl.ds(row0_tile, _G)], ref.at[0], sem_st).wait()

        def scale(gbuf, grow, p1):
            @plsc.parallel_loop(0, _E // 16, 1, unroll=2)
            def sbody(m):
                vv = valsS[gbuf, grow, pl.ds(m * 16, 16)]
                for el in range(16):
                    e = m * 16 + el
                    vsc = vv[el]
                    for q in range(4):
                        fsl = pl.ds(q * 16, 16)
                        gath[p1, e, fsl] = gath[p1, e, fsl] * vsc

        stage_fire(0, row0_tile)
        stage_drain()
        transform(0, 0, 0)
        start_gather(0)

        def body(b, _):
            j0row = row0_tile + b * 8
            gd = {}
            sd = {}
            for k in range(8):
                p = k % 2
                gbuf, grow = k // 4, k % 4
                nk = k + 1
                ntb, ntr, np_ = (nk // 4) % 2, nk % 4, nk % 2
                if k == 0:
                    wait_gather_rec(0)
                else:
                    gd[k].wait()
                transform(ntb, ntr, np_)
                if k == 0:
                    @pl.when(b > 0)
                    def _():
                        wait_scatter_rec(1)
                else:
                    sd[k - 1].wait()
                gd[nk] = start_gather(np_)
                if k == 0:
                    stage_fire(1, j0row + 4)
                elif k == 2 or k == 6:
                    stage_drain()
                elif k == 4:
                    stage_fire(0, j0row + 8)
                scale(gbuf, grow, p)
                sd[k] = start_scatter(gbuf, grow, p)
            return 0

        lax.fori_loop(0, _NB, body, 0)
        wait_gather_rec(0)
        wait_scatter_rec(1)
        plsc.subcore_barrier()

        @pl.when(s < 15)
        def _():
            for j in range(_RPT // _ZCH):
                src = acc.at[pl.ds(s * _RPT + j * _ZCH, _ZCH)]
                pltpu.sync_copy(
                    src, out.at[pl.ds(c * _VP + s * _RPT + j * _ZCH, _ZCH)])

        @pl.when(s == 15)
        def _():
            for j in range((_VP - 15 * _RPT) // _ZCH):
                src = acc.at[pl.ds(s * _RPT + j * _ZCH, _ZCH)]
                pltpu.sync_copy(
                    src, out.at[pl.ds(c * _VP + s * _RPT + j * _ZCH, _ZCH)])

        plsc.subcore_barrier()

    one_pass(x0i, x1h, 2, c)
    one_pass(x1h, y2h, 1, c * _VP)


def _sc_lap(x0i, cols2, rows2, vals2):
    mesh = plsc.VectorSubcoreMesh(core_axis_name="c", subcore_axis_name="s")
    f = pl.kernel(
        _sc_body,
        out_type=[
            jax.ShapeDtypeStruct((2 * _VP, _HALF), jnp.float32),
            jax.ShapeDtypeStruct((2 * _VP, _HALF), jnp.float32),
        ],
        mesh=mesh,
        scratch_types=[
            pltpu.VMEM((2, _G, _E), jnp.int32),
            pltpu.VMEM((2, _G, _E), jnp.int32),
            pltpu.VMEM((2, _G, _E), jnp.float32),
            pltpu.VMEM((2, _E), jnp.int32),
            pltpu.VMEM((2, _E, _HALF), jnp.float32),
            pltpu.VMEM((_ZB, _HALF), jnp.float32),
            pltpu.VMEM_SHARED((_RPAD, _HALF), jnp.float32),
            pltpu.SemaphoreType.DMA,
            pltpu.SemaphoreType.DMA,
            pltpu.SemaphoreType.DMA,
            pltpu.SemaphoreType.DMA,
            pltpu.SemaphoreType.DMA,
        ],
        compiler_params=pltpu.CompilerParams(use_tc_tiling_on_sc=False),
    )
    return f(x0i, cols2, rows2, vals2)


def _mm_body(x0_ref, x1_ref, y2_ref, wa_ref, wb_ref, b_ref, o_ref):
    acc = jnp.dot(x0_ref[...], wa_ref[...], preferred_element_type=jnp.float32)
    acc += jnp.dot(x1_ref[0], wb_ref[0], preferred_element_type=jnp.float32)
    acc += jnp.dot(x1_ref[1], wb_ref[1], preferred_element_type=jnp.float32)
    acc += jnp.dot(y2_ref[0], wb_ref[2], preferred_element_type=jnp.float32)
    acc += jnp.dot(y2_ref[1], wb_ref[3], preferred_element_type=jnp.float32)
    o_ref[...] = acc + b_ref[...]


def _mm(x0, x1h, y2h, wa, wb, bias2):
    vb = 1000
    return pl.pallas_call(
        _mm_body,
        grid=(_VP // vb,),
        in_specs=[
            pl.BlockSpec((vb, _F), lambda i: (i, 0)),
            pl.BlockSpec((2, vb, _HALF), lambda i: (0, i, 0)),
            pl.BlockSpec((2, vb, _HALF), lambda i: (0, i, 0)),
            pl.BlockSpec((_F, _FO), lambda i: (0, 0)),
            pl.BlockSpec((4, _HALF, _FO), lambda i: (0, 0, 0)),
            pl.BlockSpec((1, _FO), lambda i: (0, 0)),
        ],
        out_specs=pl.BlockSpec((vb, _FO), lambda i: (i, 0)),
        out_shape=jax.ShapeDtypeStruct((_VP, _FO), jnp.float32),
    )(x0, x1h, y2h, wa, wb, bias2)


@jax.jit
def kernel(x, lap_indices, lap_values, weight, bias):
    x4 = x.reshape(_VP, _POOL, _F)
    x0 = _pool(x4)
    x0i = x0.reshape(2 * _VP, _HALF)

    npad = _NR * _E - _NNZ
    rows2 = jnp.pad(lap_indices[0], (0, npad)).reshape(_NR, _E)
    cols2 = jnp.pad(lap_indices[1], (0, npad)).reshape(_NR, _E)
    vals2 = jnp.pad(lap_values, (0, npad)).reshape(_NR, _E)
    x1h, y2h = _sc_lap(x0i, cols2, rows2, vals2)

    w3 = weight.reshape(_F, _K, _FO).transpose(1, 0, 2)
    wa = w3[0] - w3[2]
    wb = jnp.stack([w3[1, :_HALF], w3[1, _HALF:],
                    2.0 * w3[2, :_HALF], 2.0 * w3[2, _HALF:]])
    out = _mm(x0, x1h.reshape(2, _VP, _HALF), y2h.reshape(2, _VP, _HALF),
              wa, wb, bias.reshape(1, _FO))
    return out.reshape(1, _VP, _FO)


# speedup vs baseline: 1.0579x; 1.0579x over previous
import jax
import jax.numpy as jnp
from jax import lax
from jax.experimental import pallas as pl
from jax.experimental.pallas import tpu as pltpu
from jax.experimental.pallas import tpu_sc as plsc

_V = 100000
_F = 128
_FO = 128
_K = 3
_POOL = 4
_VP = _V // _POOL
_NNZ = 600000
_HALF = _F // 2

_E = 128
_RT = 296
_NNZ_PAD = 16 * _RT * _E
_G = 4
_NB = _RT // 8
_NR = 16 * _RT + _G

_RPAD = 25600
_RPT = _RPAD // 16
_ZCH = 200
_ZB = 40


def _pool_body(x_ref, o_ref):
    o_ref[...] = jnp.mean(x_ref[...], axis=1)


def _pool(x4):
    vb = 1000
    return pl.pallas_call(
        _pool_body,
        grid=(_VP // vb,),
        in_specs=[pl.BlockSpec((vb, _POOL, _F), lambda i: (i, 0, 0))],
        out_specs=pl.BlockSpec((vb, _F), lambda i: (i, 0)),
        out_shape=jax.ShapeDtypeStruct((_VP, _F), jnp.float32),
    )(x4)


def _sc_body(x0i, cols2, rows2, vals2, x1h, y2h,
             colsS, rowsS, valsS, idxT, gath, zbuf, acc,
             sem_st, sg0, sg1, ss0, ss1):
    c = lax.axis_index("c")
    s = lax.axis_index("s")
    sg = [sg0, sg1]
    ss = [ss0, ss1]
    row0_tile = s * _RT

    def zero_zbuf(i, _):
        for q in range(4):
            zbuf[i, pl.ds(q * 16, 16)] = jnp.zeros((16,), jnp.float32)
        return 0

    lax.fori_loop(0, _ZB, zero_zbuf, 0)

    def one_pass(table, out, mul, off):
        def zloop(j, _):
            pltpu.sync_copy(zbuf, acc.at[pl.ds(s * _RPT + j * _ZB, _ZB)])
            return 0

        lax.fori_loop(0, _RPT // _ZB, zloop, 0)
        plsc.subcore_barrier()

        def transform(gbuf, grow, p1):
            for q in range(_E // 16):
                sl = pl.ds(q * 16, 16)
                if mul == 2:
                    idxT[p1, sl] = colsS[gbuf, grow, sl] * 2 + off
                else:
                    idxT[p1, sl] = colsS[gbuf, grow, sl] + off

        def start_gather(p1):
            return pltpu.async_copy(table.at[idxT.at[p1]], gath.at[p1],
                                    sg[p1])

        def wait_gather_rec(p1):
            pltpu.make_async_copy(
                table.at[pl.ds(0, _E)], gath.at[p1], sg[p1]).wait()

        def start_scatter(gbuf, grow, p1):
            return pltpu.async_copy(gath.at[p1], acc.at[rowsS.at[gbuf, grow]],
                                    ss[p1], add=True)

        def wait_scatter_rec(p1):
            pltpu.make_async_copy(
                table.at[pl.ds(0, _E)], gath.at[p1], ss[p1]).wait()

        def stage_fire(gbuf, r0):
            pltpu.async_copy(cols2.at[pl.ds(r0, _G)], colsS.at[gbuf], sem_st)
            pltpu.async_copy(rows2.at[pl.ds(r0, _G)], rowsS.at[gbuf], sem_st)
            pltpu.async_copy(vals2.at[pl.ds(r0, _G)], valsS.at[gbuf], sem_st)

        def stage_drain():
            for ref, hbm in ((colsS, cols2), (rowsS, rows2), (valsS, vals2)):
                pltpu.make_async_copy(
                    hbm.at[pl.ds(row0_tile, _G)], ref.at[0], sem_st).wait()

        def scale(gbuf, grow, p1):
            @plsc.parallel_loop(0, _E // 16, 1, unroll=2)
            def sbody(m):
                vv = valsS[gbuf, grow, pl.ds(m * 16, 16)]
                for el in range(16):
                    e = m * 16 + el
                    vsc = vv[el]
                    for q in range(4):
                        fsl = pl.ds(q * 16, 16)
                        gath[p1, e, fsl] = gath[p1, e, fsl] * vsc

        stage_fire(0, row0_tile)
        stage_drain()
        transform(0, 0, 0)
        start_gather(0)

        def body(b, _):
            j0row = row0_tile + b * 8
            gd = {}
            sd = {}
            for k in range(8):
                p = k % 2
                gbuf, grow = k // 4, k % 4
                nk = k + 1
                ntb, ntr, np_ = (nk // 4) % 2, nk % 4, nk % 2
                transform(ntb, ntr, np_)
                if k == 0:
                    @pl.when(b > 0)
                    def _():
                        wait_scatter_rec(1)
                else:
                    sd[k - 1].wait()
                gd[nk] = start_gather(np_)
                if k == 0:
                    wait_gather_rec(0)
                else:
                    gd[k].wait()
                if k == 0:
                    stage_fire(1, j0row + 4)
                elif k == 2 or k == 6:
                    stage_drain()
                elif k == 4:
                    stage_fire(0, j0row + 8)
                scale(gbuf, grow, p)
                sd[k] = start_scatter(gbuf, grow, p)
            return 0

        lax.fori_loop(0, _NB, body, 0)
        wait_gather_rec(0)
        wait_scatter_rec(1)
        plsc.subcore_barrier()

        @pl.when(s < 15)
        def _():
            for j in range(_RPT // _ZCH):
                src = acc.at[pl.ds(s * _RPT + j * _ZCH, _ZCH)]
                pltpu.sync_copy(
                    src, out.at[pl.ds(c * _VP + s * _RPT + j * _ZCH, _ZCH)])

        @pl.when(s == 15)
        def _():
            for j in range((_VP - 15 * _RPT) // _ZCH):
                src = acc.at[pl.ds(s * _RPT + j * _ZCH, _ZCH)]
                pltpu.sync_copy(
                    src, out.at[pl.ds(c * _VP + s * _RPT + j * _ZCH, _ZCH)])

        plsc.subcore_barrier()

    one_pass(x0i, x1h, 2, c)
    one_pass(x1h, y2h, 1, c * _VP)


def _sc_lap(x0i, cols2, rows2, vals2):
    mesh = plsc.VectorSubcoreMesh(core_axis_name="c", subcore_axis_name="s")
    f = pl.kernel(
        _sc_body,
        out_type=[
            jax.ShapeDtypeStruct((2 * _VP, _HALF), jnp.float32),
            jax.ShapeDtypeStruct((2 * _VP, _HALF), jnp.float32),
        ],
        mesh=mesh,
        scratch_types=[
            pltpu.VMEM((2, _G, _E), jnp.int32),
            pltpu.VMEM((2, _G, _E), jnp.int32),
            pltpu.VMEM((2, _G, _E), jnp.float32),
            pltpu.VMEM((2, _E), jnp.int32),
            pltpu.VMEM((2, _E, _HALF), jnp.float32),
            pltpu.VMEM((_ZB, _HALF), jnp.float32),
            pltpu.VMEM_SHARED((_RPAD, _HALF), jnp.float32),
            pltpu.SemaphoreType.DMA,
            pltpu.SemaphoreType.DMA,
            pltpu.SemaphoreType.DMA,
            pltpu.SemaphoreType.DMA,
            pltpu.SemaphoreType.DMA,
        ],
        compiler_params=pltpu.CompilerParams(use_tc_tiling_on_sc=False),
    )
    return f(x0i, cols2, rows2, vals2)


def _mm_body(x0_ref, x1_ref, y2_ref, wa_ref, wb_ref, b_ref, o_ref):
    acc = jnp.dot(x0_ref[...], wa_ref[...], preferred_element_type=jnp.float32)
    acc += jnp.dot(x1_ref[0], wb_ref[0], preferred_element_type=jnp.float32)
    acc += jnp.dot(x1_ref[1], wb_ref[1], preferred_element_type=jnp.float32)
    acc += jnp.dot(y2_ref[0], wb_ref[2], preferred_element_type=jnp.float32)
    acc += jnp.dot(y2_ref[1], wb_ref[3], preferred_element_type=jnp.float32)
    o_ref[...] = acc + b_ref[...]


def _mm(x0, x1h, y2h, wa, wb, bias2):
    vb = 1000
    return pl.pallas_call(
        _mm_body,
        grid=(_VP // vb,),
        in_specs=[
            pl.BlockSpec((vb, _F), lambda i: (i, 0)),
            pl.BlockSpec((2, vb, _HALF), lambda i: (0, i, 0)),
            pl.BlockSpec((2, vb, _HALF), lambda i: (0, i, 0)),
            pl.BlockSpec((_F, _FO), lambda i: (0, 0)),
            pl.BlockSpec((4, _HALF, _FO), lambda i: (0, 0, 0)),
            pl.BlockSpec((1, _FO), lambda i: (0, 0)),
        ],
        out_specs=pl.BlockSpec((vb, _FO), lambda i: (i, 0)),
        out_shape=jax.ShapeDtypeStruct((_VP, _FO), jnp.float32),
    )(x0, x1h, y2h, wa, wb, bias2)


@jax.jit
def kernel(x, lap_indices, lap_values, weight, bias):
    x4 = x.reshape(_VP, _POOL, _F)
    x0 = _pool(x4)
    x0i = x0.reshape(2 * _VP, _HALF)

    npad = _NR * _E - _NNZ
    rows2 = jnp.pad(lap_indices[0], (0, npad)).reshape(_NR, _E)
    cols2 = jnp.pad(lap_indices[1], (0, npad)).reshape(_NR, _E)
    vals2 = jnp.pad(lap_values, (0, npad)).reshape(_NR, _E)
    x1h, y2h = _sc_lap(x0i, cols2, rows2, vals2)

    w3 = weight.reshape(_F, _K, _FO).transpose(1, 0, 2)
    wa = w3[0] - w3[2]
    wb = jnp.stack([w3[1, :_HALF], w3[1, _HALF:],
                    2.0 * w3[2, :_HALF], 2.0 * w3[2, _HALF:]])
    out = _mm(x0, x1h.reshape(2, _VP, _HALF), y2h.reshape(2, _VP, _HALF),
              wa, wb, bias.reshape(1, _FO))
    return out.reshape(1, _VP, _FO)


# speedup vs baseline: 1.3433x; 1.2698x over previous
import jax
import jax.numpy as jnp
from jax import lax
from jax.experimental import pallas as pl
from jax.experimental.pallas import tpu as pltpu
from jax.experimental.pallas import tpu_sc as plsc

_V = 100000
_F = 128
_FO = 128
_K = 3
_POOL = 4
_VP = _V // _POOL
_NNZ = 600000
_HALF = _F // 2

_E = 128
_NNZ_PAD = 600064
_NR = _NNZ_PAD // _E
_RT = _NR // 16
_G = 8
_NG = _RT // _G
_TAIL = _RT - _NG * _G

_RPAD = 25600
_RPT = _RPAD // 16
_ZCH = 200
_ZB = 40


def _pool_body(x_ref, o_ref):
    o_ref[...] = jnp.mean(x_ref[...], axis=1)


def _pool(x4):
    vb = 1000
    return pl.pallas_call(
        _pool_body,
        grid=(_VP // vb,),
        in_specs=[pl.BlockSpec((vb, _POOL, _F), lambda i: (i, 0, 0))],
        out_specs=pl.BlockSpec((vb, _F), lambda i: (i, 0)),
        out_shape=jax.ShapeDtypeStruct((_VP, _F), jnp.float32),
    )(x4)


def _sc_body(x0i, cols2, rows2, vals2, x1h, y2h,
             colsS, rowsS, valsS, idxT, gath, zbuf, acc,
             sem_st, sg0, sg1, ss0, ss1):
    c = lax.axis_index("c")
    s = lax.axis_index("s")
    sg = [sg0, sg1]
    ss = [ss0, ss1]
    row0_tile = s * _RT

    def zero_zbuf(i, _):
        for q in range(4):
            zbuf[i, pl.ds(q * 16, 16)] = jnp.zeros((16,), jnp.float32)
        return 0

    lax.fori_loop(0, _ZB, zero_zbuf, 0)

    def one_pass(table, out, mul, off):
        def zloop(j, _):
            pltpu.sync_copy(zbuf, acc.at[pl.ds(s * _RPT + j * _ZB, _ZB)])
            return 0

        lax.fori_loop(0, _RPT // _ZB, zloop, 0)
        plsc.subcore_barrier()

        def transform(k):
            p = k % 2
            for q in range(_E // 16):
                sl = pl.ds(q * 16, 16)
                if mul == 2:
                    idxT[p, sl] = colsS[k, sl] * 2 + off
                else:
                    idxT[p, sl] = colsS[k, sl] + off

        def start_gather(k):
            p = k % 2
            return pltpu.async_copy(table.at[idxT.at[p]], gath.at[p], sg[p])

        def scale(k):
            p = k % 2

            @plsc.parallel_loop(0, _E // 16, 1, unroll=2)
            def sbody(m):
                vv = valsS[k, pl.ds(m * 16, 16)]
                for el in range(16):
                    e = m * 16 + el
                    sp = vv[el]
                    for q in range(4):
                        fsl = pl.ds(q * 16, 16)
                        gath[p, e, fsl] = gath[p, e, fsl] * sp

        def start_scatter(k):
            p = k % 2
            return pltpu.async_copy(gath.at[p], acc.at[rowsS.at[k]],
                                    ss[p], add=True)

        def group(row0, nrows):
            d1 = pltpu.async_copy(cols2.at[pl.ds(row0, nrows)],
                                  colsS.at[pl.ds(0, nrows)], sem_st)
            d2 = pltpu.async_copy(rows2.at[pl.ds(row0, nrows)],
                                  rowsS.at[pl.ds(0, nrows)], sem_st)
            d3 = pltpu.async_copy(vals2.at[pl.ds(row0, nrows)],
                                  valsS.at[pl.ds(0, nrows)], sem_st)
            d1.wait(); d2.wait(); d3.wait()

            gd = [None, None]
            sd = [None, None]
            transform(0)
            gd[0] = start_gather(0)
            for k in range(nrows):
                p = k % 2
                if k + 1 < nrows:
                    transform(k + 1)
                    if k >= 1:
                        sd[1 - p].wait()
                    gd[1 - p] = start_gather(k + 1)
                gd[p].wait()
                scale(k)
                sd[p] = start_scatter(k)
            if nrows >= 2:
                sd[nrows % 2].wait()
            sd[(nrows - 1) % 2].wait()

        def grp_body(g, _):
            group(row0_tile + g * _G, _G)
            return 0

        lax.fori_loop(0, _NG, grp_body, 0)
        group(row0_tile + _NG * _G, _TAIL)
        plsc.subcore_barrier()

        @pl.when(s < 15)
        def _():
            for j in range(_RPT // _ZCH):
                src = acc.at[pl.ds(s * _RPT + j * _ZCH, _ZCH)]
                pltpu.sync_copy(
                    src, out.at[pl.ds(c * _VP + s * _RPT + j * _ZCH, _ZCH)])

        @pl.when(s == 15)
        def _():
            for j in range((_VP - 15 * _RPT) // _ZCH):
                src = acc.at[pl.ds(s * _RPT + j * _ZCH, _ZCH)]
                pltpu.sync_copy(
                    src, out.at[pl.ds(c * _VP + s * _RPT + j * _ZCH, _ZCH)])

        plsc.subcore_barrier()

    one_pass(x0i, x1h, 2, c)
    one_pass(x1h, y2h, 1, c * _VP)


def _sc_lap(x0i, cols2, rows2, vals2):
    mesh = plsc.VectorSubcoreMesh(core_axis_name="c", subcore_axis_name="s")
    f = pl.kernel(
        _sc_body,
        out_type=[
            jax.ShapeDtypeStruct((2 * _VP, _HALF), jnp.float32),
            jax.ShapeDtypeStruct((2 * _VP, _HALF), jnp.float32),
        ],
        mesh=mesh,
        scratch_types=[
            pltpu.VMEM((_G, _E), jnp.int32),
            pltpu.VMEM((_G, _E), jnp.int32),
            pltpu.VMEM((_G, _E), jnp.float32),
            pltpu.VMEM((2, _E), jnp.int32),
            pltpu.VMEM((2, _E, _HALF), jnp.float32),
            pltpu.VMEM((_ZB, _HALF), jnp.float32),
            pltpu.VMEM_SHARED((_RPAD, _HALF), jnp.float32),
            pltpu.SemaphoreType.DMA,
            pltpu.SemaphoreType.DMA,
            pltpu.SemaphoreType.DMA,
            pltpu.SemaphoreType.DMA,
            pltpu.SemaphoreType.DMA,
        ],
        compiler_params=pltpu.CompilerParams(use_tc_tiling_on_sc=False),
    )
    return f(x0i, cols2, rows2, vals2)


def _mm_body(x0_ref, x1_ref, y2_ref, wa_ref, wb_ref, b_ref, o_ref):
    acc = jnp.dot(x0_ref[...], wa_ref[...], preferred_element_type=jnp.float32)
    acc += jnp.dot(x1_ref[0], wb_ref[0], preferred_element_type=jnp.float32)
    acc += jnp.dot(x1_ref[1], wb_ref[1], preferred_element_type=jnp.float32)
    acc += jnp.dot(y2_ref[0], wb_ref[2], preferred_element_type=jnp.float32)
    acc += jnp.dot(y2_ref[1], wb_ref[3], preferred_element_type=jnp.float32)
    o_ref[...] = acc + b_ref[...]


def _mm(x0, x1h, y2h, wa, wb, bias2):
    vb = 1000
    return pl.pallas_call(
        _mm_body,
        grid=(_VP // vb,),
        in_specs=[
            pl.BlockSpec((vb, _F), lambda i: (i, 0)),
            pl.BlockSpec((2, vb, _HALF), lambda i: (0, i, 0)),
            pl.BlockSpec((2, vb, _HALF), lambda i: (0, i, 0)),
            pl.BlockSpec((_F, _FO), lambda i: (0, 0)),
            pl.BlockSpec((4, _HALF, _FO), lambda i: (0, 0, 0)),
            pl.BlockSpec((1, _FO), lambda i: (0, 0)),
        ],
        out_specs=pl.BlockSpec((vb, _FO), lambda i: (i, 0)),
        out_shape=jax.ShapeDtypeStruct((_VP, _FO), jnp.float32),
    )(x0, x1h, y2h, wa, wb, bias2)


@jax.jit
def kernel(x, lap_indices, lap_values, weight, bias):
    x4 = x.reshape(_VP, _POOL, _F)
    x0 = _pool(x4)
    x0i = x0.reshape(2 * _VP, _HALF)

    npad = _NNZ_PAD - _NNZ
    rows2 = jnp.pad(lap_indices[0], (0, npad)).reshape(_NR, _E)
    cols2 = jnp.pad(lap_indices[1], (0, npad)).reshape(_NR, _E)
    vals2 = jnp.pad(lap_values, (0, npad)).reshape(_NR, _E)
    x1h, y2h = _sc_lap(x0i, cols2, rows2, vals2)

    w3 = weight.reshape(_F, _K, _FO).transpose(1, 0, 2)
    wa = w3[0] - w3[2]
    wb = jnp.stack([w3[1, :_HALF], w3[1, _HALF:],
                    2.0 * w3[2, :_HALF], 2.0 * w3[2, _HALF:]])
    out = _mm(x0, x1h.reshape(2, _VP, _HALF), y2h.reshape(2, _VP, _HALF),
              wa, wb, bias.reshape(1, _FO))
    return out.reshape(1, _VP, _FO)


# speedup vs baseline: 1.4150x; 1.0534x over previous
import jax
import jax.numpy as jnp
from jax import lax
from jax.experimental import pallas as pl
from jax.experimental.pallas import tpu as pltpu
from jax.experimental.pallas import tpu_sc as plsc

_V = 100000
_F = 128
_FO = 128
_K = 3
_POOL = 4
_VP = _V // _POOL
_NNZ = 600000
_HALF = _F // 2

_E = 128
_NNZ_PAD = 600064
_NR = _NNZ_PAD // _E
_RT = _NR // 16
_G = 8
_NG = _RT // _G
_TAIL = _RT - _NG * _G

_RPAD = 25280
_RPT = _RPAD // 16
_ZCH = 158
_ZCL = 130
_ZB = 20


def _pool_body(x_ref, o_ref):
    o_ref[...] = jnp.mean(x_ref[...], axis=1)


def _pool(x4):
    vb = 1000
    return pl.pallas_call(
        _pool_body,
        grid=(_VP // vb,),
        in_specs=[pl.BlockSpec((vb, _POOL, _F), lambda i: (i, 0, 0))],
        out_specs=pl.BlockSpec((vb, _F), lambda i: (i, 0)),
        out_shape=jax.ShapeDtypeStruct((_VP, _F), jnp.float32),
    )(x4)


def _sc_body(x0i, cols2, rows2, vals2, x1h, y2h,
             colsS, rowsS, valsS, idxT, gath, zbuf, acc,
             sem_st, sg0, sg1, sg2, ss0, ss1, ss2):
    c = lax.axis_index("c")
    s = lax.axis_index("s")
    sg = [sg0, sg1, sg2]
    ss = [ss0, ss1, ss2]
    row0_tile = s * _RT

    def zero_zbuf(i, _):
        for q in range(4):
            zbuf[i, pl.ds(q * 16, 16)] = jnp.zeros((16,), jnp.float32)
        return 0

    lax.fori_loop(0, _ZB, zero_zbuf, 0)

    def zero_acc():
        def zloop(j, _):
            pltpu.sync_copy(zbuf, acc.at[pl.ds(s * _RPT + j * _ZB, _ZB)])
            return 0

        lax.fori_loop(0, _RPT // _ZB, zloop, 0)

    def one_pass(table, out, mul, off, zero_next):
        def transform(k):
            p = k % 3
            for q in range(_E // 16):
                sl = pl.ds(q * 16, 16)
                if mul == 2:
                    idxT[p, sl] = colsS[k, sl] * 2 + off
                else:
                    idxT[p, sl] = colsS[k, sl] + off

        def start_gather(k):
            p = k % 3
            return pltpu.async_copy(table.at[idxT.at[p]], gath.at[p], sg[p])

        def scale(k):
            p = k % 3

            @plsc.parallel_loop(0, _E // 16, 1, unroll=2)
            def sbody(m):
                vv = valsS[k, pl.ds(m * 16, 16)]
                for el in range(16):
                    e = m * 16 + el
                    sp = vv[el]
                    for q in range(4):
                        fsl = pl.ds(q * 16, 16)
                        gath[p, e, fsl] = gath[p, e, fsl] * sp

        def start_scatter(k):
            p = k % 3
            return pltpu.async_copy(gath.at[p], acc.at[rowsS.at[k]],
                                    ss[p], add=True)

        def group(row0, nrows):
            d1 = pltpu.async_copy(cols2.at[pl.ds(row0, nrows)],
                                  colsS.at[pl.ds(0, nrows)], sem_st)
            d2 = pltpu.async_copy(rows2.at[pl.ds(row0, nrows)],
                                  rowsS.at[pl.ds(0, nrows)], sem_st)
            d3 = pltpu.async_copy(vals2.at[pl.ds(row0, nrows)],
                                  valsS.at[pl.ds(0, nrows)], sem_st)
            d1.wait(); d2.wait(); d3.wait()

            gd = [None, None, None]
            sd = [None, None, None]
            transform(0)
            gd[0] = start_gather(0)
            transform(1)
            gd[1] = start_gather(1)
            for k in range(nrows):
                p = k % 3
                if k + 2 < nrows:
                    transform(k + 2)
                    if k >= 1:
                        sd[(k + 2) % 3].wait()
                    gd[(k + 2) % 3] = start_gather(k + 2)
                gd[p].wait()
                scale(k)
                sd[p] = start_scatter(k)
            for k in range(max(0, nrows - 3), nrows):
                sd[k % 3].wait()

        def grp_body(g, _):
            group(row0_tile + g * _G, _G)
            return 0

        lax.fori_loop(0, _NG, grp_body, 0)
        group(row0_tile + _NG * _G, _TAIL)
        plsc.subcore_barrier()

        @pl.when(s < 15)
        def _():
            for j in range(_RPT // _ZCH):
                src = acc.at[pl.ds(s * _RPT + j * _ZCH, _ZCH)]
                pltpu.sync_copy(
                    src, out.at[pl.ds(c * _VP + s * _RPT + j * _ZCH, _ZCH)])

        @pl.when(s == 15)
        def _():
            for j in range((_VP - 15 * _RPT) // _ZCL):
                src = acc.at[pl.ds(s * _RPT + j * _ZCL, _ZCL)]
                pltpu.sync_copy(
                    src, out.at[pl.ds(c * _VP + s * _RPT + j * _ZCL, _ZCL)])

        if zero_next:
            zero_acc()
        plsc.subcore_barrier()

    zero_acc()
    plsc.subcore_barrier()
    one_pass(x0i, x1h, 2, c, True)
    one_pass(x1h, y2h, 1, c * _VP, False)


def _sc_lap(x0i, cols2, rows2, vals2):
    mesh = plsc.VectorSubcoreMesh(core_axis_name="c", subcore_axis_name="s")
    f = pl.kernel(
        _sc_body,
        out_type=[
            jax.ShapeDtypeStruct((2 * _VP, _HALF), jnp.float32),
            jax.ShapeDtypeStruct((2 * _VP, _HALF), jnp.float32),
        ],
        mesh=mesh,
        scratch_types=[
            pltpu.VMEM((_G, _E), jnp.int32),
            pltpu.VMEM((_G, _E), jnp.int32),
            pltpu.VMEM((_G, _E), jnp.float32),
            pltpu.VMEM((3, _E), jnp.int32),
            pltpu.VMEM((3, _E, _HALF), jnp.float32),
            pltpu.VMEM((_ZB, _HALF), jnp.float32),
            pltpu.VMEM_SHARED((_RPAD, _HALF), jnp.float32),
            pltpu.SemaphoreType.DMA,
            pltpu.SemaphoreType.DMA,
            pltpu.SemaphoreType.DMA,
            pltpu.SemaphoreType.DMA,
            pltpu.SemaphoreType.DMA,
            pltpu.SemaphoreType.DMA,
            pltpu.SemaphoreType.DMA,
        ],
        compiler_params=pltpu.CompilerParams(use_tc_tiling_on_sc=False),
    )
    return f(x0i, cols2, rows2, vals2)


def _mm_body(x0_ref, x1_ref, y2_ref, wa_ref, wb_ref, b_ref, o_ref):
    acc = jnp.dot(x0_ref[...], wa_ref[...], preferred_element_type=jnp.float32)
    acc += jnp.dot(x1_ref[0], wb_ref[0], preferred_element_type=jnp.float32)
    acc += jnp.dot(x1_ref[1], wb_ref[1], preferred_element_type=jnp.float32)
    acc += jnp.dot(y2_ref[0], wb_ref[2], preferred_element_type=jnp.float32)
    acc += jnp.dot(y2_ref[1], wb_ref[3], preferred_element_type=jnp.float32)
    o_ref[...] = acc + b_ref[...]


def _mm(x0, x1h, y2h, wa, wb, bias2):
    vb = 1000
    return pl.pallas_call(
        _mm_body,
        grid=(_VP // vb,),
        in_specs=[
            pl.BlockSpec((vb, _F), lambda i: (i, 0)),
            pl.BlockSpec((2, vb, _HALF), lambda i: (0, i, 0)),
            pl.BlockSpec((2, vb, _HALF), lambda i: (0, i, 0)),
            pl.BlockSpec((_F, _FO), lambda i: (0, 0)),
            pl.BlockSpec((4, _HALF, _FO), lambda i: (0, 0, 0)),
            pl.BlockSpec((1, _FO), lambda i: (0, 0)),
        ],
        out_specs=pl.BlockSpec((vb, _FO), lambda i: (i, 0)),
        out_shape=jax.ShapeDtypeStruct((_VP, _FO), jnp.float32),
    )(x0, x1h, y2h, wa, wb, bias2)


@jax.jit
def kernel(x, lap_indices, lap_values, weight, bias):
    x4 = x.reshape(_VP, _POOL, _F)
    x0 = _pool(x4)
    x0i = x0.reshape(2 * _VP, _HALF)

    npad = _NNZ_PAD - _NNZ
    rows2 = jnp.pad(lap_indices[0], (0, npad)).reshape(_NR, _E)
    cols2 = jnp.pad(lap_indices[1], (0, npad)).reshape(_NR, _E)
    vals2 = jnp.pad(lap_values, (0, npad)).reshape(_NR, _E)
    x1h, y2h = _sc_lap(x0i, cols2, rows2, vals2)

    w3 = weight.reshape(_F, _K, _FO).transpose(1, 0, 2)
    wa = w3[0] - w3[2]
    wb = jnp.stack([w3[1, :_HALF], w3[1, _HALF:],
                    2.0 * w3[2, :_HALF], 2.0 * w3[2, _HALF:]])
    out = _mm(x0, x1h.reshape(2, _VP, _HALF), y2h.reshape(2, _VP, _HALF),
              wa, wb, bias.reshape(1, _FO))
    return out.reshape(1, _VP, _FO)
